# Initial kernel scaffold; baseline (speedup 1.0000x reference)
import jax
import jax.numpy as jnp
from jax.experimental import pallas as pl


def kernel(x, table):
    raise NotImplementedError("write your pallas kernel here")


# speedup vs baseline: 1.8451x; 1.8451x over previous
import functools

import jax
import jax.numpy as jnp
from jax import lax
from jax.experimental import pallas as pl
from jax.experimental.pallas import tpu as pltpu
from jax.experimental.pallas import tpu_sc as plsc

EMBED_DIM = 64
NUM_CORES = 2
NUM_SUBCORES = 16
NUM_WORKERS = NUM_CORES * NUM_SUBCORES
CHUNK = 1024


def _make_call(n_flat):
    assert n_flat % (NUM_WORKERS * CHUNK) == 0
    b_per_w = n_flat // NUM_WORKERS
    n_chunks = b_per_w // CHUNK
    mesh = plsc.VectorSubcoreMesh(core_axis_name="c", subcore_axis_name="s")

    @functools.partial(
        pl.kernel,
        mesh=mesh,
        compiler_params=pltpu.CompilerParams(use_tc_tiling_on_sc=False),
        out_type=jax.ShapeDtypeStruct((n_flat, EMBED_DIM), jnp.float32),
        scratch_types=[
            pltpu.VMEM((CHUNK,), jnp.int32),
            pltpu.VMEM((CHUNK, EMBED_DIM), jnp.float32),
            pltpu.SemaphoreType.DMA,
        ],
    )
    def gather_kernel(table_hbm, idx_hbm, out_hbm, idx_v, rows_v, sem):
        wid = lax.axis_index("s") * NUM_CORES + lax.axis_index("c")
        base = wid * b_per_w

        def step(g, carry):
            off = base + g * CHUNK
            pltpu.sync_copy(idx_hbm.at[pl.ds(off, CHUNK)], idx_v)
            pltpu.async_copy(table_hbm.at[idx_v], rows_v, sem).wait()
            pltpu.sync_copy(rows_v, out_hbm.at[pl.ds(off, CHUNK)])
            return carry

        lax.fori_loop(0, n_chunks, step, 0)

    return gather_kernel


@jax.jit
def kernel(x, table):
    batch, hist = x.shape
    n_flat = batch * hist
    flat_idx = x.reshape(n_flat).astype(jnp.int32)
    out = _make_call(n_flat)(table, flat_idx)
    return out.reshape(batch, hist, EMBED_DIM)


# speedup vs baseline: 1.8726x; 1.0149x over previous
import functools

import jax
import jax.numpy as jnp
from jax import lax
from jax.experimental import pallas as pl
from jax.experimental.pallas import tpu as pltpu
from jax.experimental.pallas import tpu_sc as plsc

EMBED_DIM = 64
NUM_CORES = 2
NUM_SUBCORES = 16
NUM_WORKERS = NUM_CORES * NUM_SUBCORES
CHUNK = 800
NBUF = 2


def _make_call(n_flat):
    assert n_flat % (NUM_WORKERS * CHUNK * NBUF) == 0
    b_per_w = n_flat // NUM_WORKERS
    n_chunks = b_per_w // CHUNK
    mesh = plsc.VectorSubcoreMesh(core_axis_name="c", subcore_axis_name="s")

    @functools.partial(
        pl.kernel,
        mesh=mesh,
        compiler_params=pltpu.CompilerParams(use_tc_tiling_on_sc=False),
        out_type=jax.ShapeDtypeStruct((n_flat, EMBED_DIM), jnp.float32),
        scratch_types=[
            pltpu.VMEM((NBUF, CHUNK), jnp.int32),
            pltpu.VMEM((NBUF, CHUNK, EMBED_DIM), jnp.float32),
            pltpu.SemaphoreType.DMA,
            pltpu.SemaphoreType.DMA,
            pltpu.SemaphoreType.DMA,
        ],
    )
    def gather_kernel(table_hbm, idx_hbm, out_hbm, idx_v, rows_v,
                      idx_sem, gat_sem, out_sem):
        wid = lax.axis_index("s") * NUM_CORES + lax.axis_index("c")
        base = wid * b_per_w

        for b in range(NBUF):
            pltpu.async_copy(
                idx_hbm.at[pl.ds(base + b * CHUNK, CHUNK)], idx_v.at[b],
                idx_sem)

        def outer(g, carry):
            for b in range(NBUF):
                c = g * NBUF + b
                pltpu.make_async_copy(
                    idx_hbm.at[pl.ds(base, CHUNK)], idx_v.at[b],
                    idx_sem).wait()

                @pl.when(g > 0)
                def _():
                    pltpu.make_async_copy(
                        rows_v.at[b], out_hbm.at[pl.ds(base, CHUNK)],
                        out_sem).wait()

                pltpu.async_copy(
                    table_hbm.at[idx_v.at[b]], rows_v.at[b], gat_sem).wait()

                pltpu.async_copy(
                    rows_v.at[b], out_hbm.at[pl.ds(base + c * CHUNK, CHUNK)],
                    out_sem)

                @pl.when(g < (n_chunks // NBUF) - 1)
                def _():
                    pltpu.async_copy(
                        idx_hbm.at[pl.ds(base + (c + NBUF) * CHUNK, CHUNK)],
                        idx_v.at[b], idx_sem)
            return carry

        lax.fori_loop(0, n_chunks // NBUF, outer, 0)

        for b in range(NBUF):
            pltpu.make_async_copy(
                rows_v.at[b], out_hbm.at[pl.ds(base, CHUNK)], out_sem).wait()

    return gather_kernel


@jax.jit
def kernel(x, table):
    batch, hist = x.shape
    n_flat = batch * hist
    flat_idx = x.reshape(n_flat).astype(jnp.int32)
    out = _make_call(n_flat)(table, flat_idx)
    return out.reshape(batch, hist, EMBED_DIM)
